# baseline (speedup 1.0000x reference)
import jax
import jax.numpy as jnp
from jax.experimental import pallas as pl


def _acc_kernel(z_ref, y_ref, out_ref):
    x = z_ref[:]
    y = y_ref[:]
    n = x.shape[0]
    num = jax.lax.dot_general(
        x, y,
        dimension_numbers=(((1,), (1,)), ((), ())),
        preferred_element_type=jnp.float32,
    )
    xn = jnp.sqrt(jnp.sum(x * x, axis=1))
    yn = jnp.sqrt(jnp.sum(y * y, axis=1))
    denom = jnp.maximum(xn[:, None] * yn[None, :], 1e-8)
    row = jax.lax.broadcasted_iota(jnp.int32, (n, n), 0)
    col = jax.lax.broadcasted_iota(jnp.int32, (n, n), 1)
    diag_num = jnp.sum(jnp.where(row == col, num, 0.0), axis=0, keepdims=True)
    diag_sim = diag_num / jnp.maximum(xn[None, :] * yn[None, :], 1e-8)
    beats = num > diag_sim * denom
    cnt = jnp.sum(jnp.where(beats, 1.0, 0.0), axis=0, keepdims=True)
    top1 = jnp.sum(jnp.where(cnt == 0.0, 1.0, 0.0), axis=1, keepdims=True)
    top10 = jnp.sum(jnp.where(cnt < 10.0, 1.0, 0.0), axis=1, keepdims=True)
    out_ref[...] = jnp.concatenate([top1, top10], axis=1) * (1.0 / n)


def kernel(Z, Y):
    out = pl.pallas_call(
        _acc_kernel,
        out_shape=jax.ShapeDtypeStruct((1, 2), jnp.float32),
    )(Z, Y)
    return (out[0, 0], out[0, 1])
